# Initial kernel scaffold; baseline (speedup 1.0000x reference)
import jax
import jax.numpy as jnp
from jax.experimental import pallas as pl


def kernel(x, edge_index, edge_type, W0, self_w0, b0, W1, self_w1, b1):
    raise NotImplementedError("write your pallas kernel here")


# speedup vs baseline: 22.3448x; 22.3448x over previous
import functools

import jax
import jax.numpy as jnp
from jax import lax
from jax.experimental import pallas as pl
from jax.experimental.pallas import tpu as pltpu
from jax.experimental.pallas import tpu_sc as plsc

_N = 10000
_E = 320000
_D = 128
_R = 8

_NC = 2
_NS = 16
_NW = _NC * _NS
_EPW = _E // _NW
_CH = 100
_NCHUNK = _EPW // _CH
_NPAD = 10240
_RPT = _NPAD // _NS

_BLK = 400
_GRID = _N // _BLK
_EBLK = _E // _GRID // 128


def _l0_body(x_ref, wflat_ref, wself_ref, xw_ref, self_ref):
    x = x_ref[...]
    xw_ref[...] = jnp.dot(x, wflat_ref[...], preferred_element_type=jnp.float32)
    self_ref[...] = jnp.dot(x, wself_ref[...], preferred_element_type=jnp.float32)


def _flat_body(src_ref, typ_ref, flat_ref):
    flat_ref[...] = src_ref[...] * _R + typ_ref[...]


def _tc_flat_idx(src2d, typ2d):
    return pl.pallas_call(
        _flat_body,
        out_shape=jax.ShapeDtypeStruct((_E // 128, 128), jnp.int32),
    )(src2d, typ2d)


def _mid_body(x_ref, self0_ref, parts_ref, b0_ref, wflat_ref, wself_ref,
              xw_ref, self_ref):
    h = x_ref[...] + self0_ref[...] + parts_ref[0] + parts_ref[1]
    h = jnp.maximum(h, 0.0) + b0_ref[...]
    xw_ref[...] = jnp.dot(h, wflat_ref[...], preferred_element_type=jnp.float32)
    self_ref[...] = jnp.dot(h, wself_ref[...], preferred_element_type=jnp.float32)


def _fin_body(self1_ref, parts_ref, b1_ref, o_ref):
    o_ref[...] = self1_ref[...] + parts_ref[0] + parts_ref[1] + b1_ref[...]


def _tc_layer0(x, wflat, wself):
    return pl.pallas_call(
        _l0_body,
        grid=(_GRID,),
        in_specs=[
            pl.BlockSpec((_BLK, _D), lambda i: (i, 0)),
            pl.BlockSpec((_D, _R * _D), lambda i: (0, 0)),
            pl.BlockSpec((_D, _D), lambda i: (0, 0)),
        ],
        out_specs=[
            pl.BlockSpec((_BLK, _R * _D), lambda i: (i, 0)),
            pl.BlockSpec((_BLK, _D), lambda i: (i, 0)),
        ],
        out_shape=[
            jax.ShapeDtypeStruct((_N, _R * _D), jnp.float32),
            jax.ShapeDtypeStruct((_N, _D), jnp.float32),
        ],
    )(x, wflat, wself)


def _tc_mid(x, self0, parts, b0row, wflat, wself):
    return pl.pallas_call(
        _mid_body,
        grid=(_GRID,),
        in_specs=[
            pl.BlockSpec((_BLK, _D), lambda i: (i, 0)),
            pl.BlockSpec((_BLK, _D), lambda i: (i, 0)),
            pl.BlockSpec((_NC, _BLK, _D), lambda i: (0, i, 0)),
            pl.BlockSpec((1, _D), lambda i: (0, 0)),
            pl.BlockSpec((_D, _R * _D), lambda i: (0, 0)),
            pl.BlockSpec((_D, _D), lambda i: (0, 0)),
        ],
        out_specs=[
            pl.BlockSpec((_BLK, _R * _D), lambda i: (i, 0)),
            pl.BlockSpec((_BLK, _D), lambda i: (i, 0)),
        ],
        out_shape=[
            jax.ShapeDtypeStruct((_N, _R * _D), jnp.float32),
            jax.ShapeDtypeStruct((_N, _D), jnp.float32),
        ],
    )(x, self0, parts, b0row, wflat, wself)


def _tc_final(self1, parts, b1row):
    return pl.pallas_call(
        _fin_body,
        grid=(_GRID,),
        in_specs=[
            pl.BlockSpec((_BLK, _D), lambda i: (i, 0)),
            pl.BlockSpec((_NC, _BLK, _D), lambda i: (0, i, 0)),
            pl.BlockSpec((1, _D), lambda i: (0, 0)),
        ],
        out_specs=pl.BlockSpec((_BLK, _D), lambda i: (i, 0)),
        out_shape=jax.ShapeDtypeStruct((_N, _D), jnp.float32),
    )(self1, parts, b1row)


def _make_sc_kernel():
    mesh = plsc.VectorSubcoreMesh(core_axis_name="c", subcore_axis_name="s")

    def body(table, gidx, didx, zinit, out, gidx_v, didx_v, rows, agg_s, sem):
        c = lax.axis_index("c")
        s = lax.axis_index("s")
        wid = s * _NC + c
        pltpu.sync_copy(gidx.at[wid], gidx_v)
        pltpu.sync_copy(didx.at[wid], didx_v)
        pltpu.sync_copy(zinit.at[pl.ds(s * _RPT, _RPT)],
                        agg_s.at[pl.ds(s * _RPT, _RPT)])
        plsc.subcore_barrier()

        @pl.loop(0, _NCHUNK)
        def _chunk(j):
            pltpu.async_copy(table.at[gidx_v.at[j]], rows, sem).wait()
            pltpu.sync_copy(rows, agg_s.at[didx_v.at[j]], add=True)

        plsc.subcore_barrier()
        pltpu.sync_copy(agg_s.at[pl.ds(s * _RPT, _RPT)],
                        out.at[c, pl.ds(s * _RPT, _RPT)])

    return pl.kernel(
        body,
        out_type=jax.ShapeDtypeStruct((_NC, _NPAD, _D), jnp.float32),
        mesh=mesh,
        scratch_types=[
            pltpu.VMEM((_NCHUNK, _CH), jnp.int32),
            pltpu.VMEM((_NCHUNK, _CH), jnp.int32),
            pltpu.VMEM((_CH, _D), jnp.float32),
            pltpu.VMEM_SHARED((_NPAD, _D), jnp.float32),
            pltpu.SemaphoreType.DMA,
        ],
    )


@functools.cache
def _sc_kernel_cached():
    return _make_sc_kernel()


def _sc_gather_scatter(table, gidx, didx, zinit):
    return _sc_kernel_cached()(table, gidx, didx, zinit)


def kernel(x, edge_index, edge_type, W0, self_w0, b0, W1, self_w1, b1):
    src2d = edge_index[0].reshape(_E // 128, 128)
    typ2d = edge_type.reshape(_E // 128, 128)
    w0flat = W0.transpose(1, 0, 2).reshape(_D, _R * _D)
    w1flat = W1.transpose(1, 0, 2).reshape(_D, _R * _D)
    b0row = b0.reshape(1, _D)
    b1row = b1.reshape(1, _D)
    zinit = jnp.zeros((_NPAD, _D), jnp.float32)

    xw0, self0 = _tc_layer0(x, w0flat, self_w0)
    flat2d = _tc_flat_idx(src2d, typ2d)
    gidx = flat2d.reshape(_NW, _NCHUNK, _CH)
    didx = edge_index[1].reshape(_NW, _NCHUNK, _CH)

    table0 = xw0.reshape(_N * _R, _D)
    parts0 = _sc_gather_scatter(table0, gidx, didx, zinit)

    xw1, self1 = _tc_mid(x, self0, parts0, b0row, w1flat, self_w1)
    table1 = xw1.reshape(_N * _R, _D)
    parts1 = _sc_gather_scatter(table1, gidx, didx, zinit)

    return _tc_final(self1, parts1, b1row)
